# baseline (device time: 74459 ns/iter reference)
import jax
import jax.numpy as jnp
from jax import lax
from jax.experimental import pallas as pl
from jax.experimental.pallas import tpu as pltpu

N_DEV = 16
N_Z = 4
N_C = 4


def kernel(x, w_mat, scale_x, scale_w):
    m_per, k = x.shape
    n_loc = w_mat.shape[1]
    half_k = k // 2

    def body(x_ref, w_ref, sx_ref, sw_ref, out_ref,
             wcomm, xb_ref, a2a_src, a2a_dst,
             col_send, col_recv,
             cwf_send, cwf_recv, ccwf_send, ccwf_recv,
             cwh_send, cwh_recv, ccwh_send, ccwh_recv,
             a2a_send_sems, a2a_recv_sems):
        my = lax.axis_index("i")
        z = my // N_C
        c = lax.rem(my, N_C)
        pb = my - c
        cw_n = pb + lax.rem(c + 1, N_C)
        ccw_n = pb + lax.rem(c + 3, N_C)

        def plane_mesh(dc, zp):
            return N_C * zp + lax.rem(c + dc + N_C, N_C)

        def full(o):
            return wcomm.at[o]

        def half(o, hb):
            return wcomm.at[o, pl.ds(hb * half_k, half_k), :]

        def mk(ref, ssem, rsem, tgt):
            return pltpu.make_async_remote_copy(
                src_ref=ref, dst_ref=ref, send_sem=ssem, recv_sem=rsem,
                device_id=(tgt,), device_id_type=pl.DeviceIdType.MESH,
            )

        barrier_sem = pltpu.get_barrier_semaphore()
        for t in range(N_DEV):
            pl.semaphore_signal(
                barrier_sem, inc=1,
                device_id=(t,), device_id_type=pl.DeviceIdType.MESH,
            )
        pl.semaphore_wait(barrier_sem, N_DEV)

        wcomm[my] = w_ref[...].astype(jnp.float8_e5m2)
        for dz in (1, 2, 3):
            zt = lax.rem(z + dz, N_Z)
            mk(full(my), col_send.at[zt], col_recv.at[z], c + N_C * zt).start()
        mk(full(my), cwf_send.at[z], cwf_recv.at[z], cw_n).start()
        mk(full(my), ccwf_send.at[z], ccwf_recv.at[z], ccw_n).start()

        xb_ref[...] = x_ref[...].astype(jnp.bfloat16)
        scale = sx_ref[0] * sw_ref[0]

        def block(origin):
            wb = wcomm[origin].astype(jnp.bfloat16)
            acc = lax.dot_general(
                xb_ref[...], wb,
                (((1,), (0,)), ((), ())),
                preferred_element_type=jnp.float32,
            )
            y = acc * scale
            yc = jnp.clip(y, -60.0, 60.0)
            return y / (1.0 + jnp.exp(-yc))

        def send_block(o):
            a2a_src[o] = block(o).astype(jnp.bfloat16)
            pltpu.make_async_remote_copy(
                src_ref=a2a_src.at[o], dst_ref=a2a_dst.at[my],
                send_sem=a2a_send_sems.at[o], recv_sem=a2a_recv_sems.at[my],
                device_id=(o,), device_id_type=pl.DeviceIdType.MESH,
            ).start()

        out_ref[pl.ds(my * m_per, m_per), :] = block(my)

        for dz in range(N_Z):
            zp = lax.rem(z + dz, N_Z)
            todo = []
            if dz >= 1:
                o = c + N_C * zp
                mk(full(o), col_send.at[zp], col_recv.at[zp], my).wait_recv()
                mk(full(o), cwf_send.at[zp], cwf_recv.at[zp], cw_n).start()
                mk(full(o), ccwf_send.at[zp], ccwf_recv.at[zp], ccw_n).start()
                todo.append(o)
            o = plane_mesh(-1, zp)
            mk(full(o), cwf_send.at[zp], cwf_recv.at[zp], my).wait_recv()
            mk(half(o, 0), cwh_send.at[zp], cwh_recv.at[zp], cw_n).start()
            todo.append(o)
            o = plane_mesh(1, zp)
            mk(full(o), ccwf_send.at[zp], ccwf_recv.at[zp], my).wait_recv()
            mk(half(o, 1), ccwh_send.at[zp], ccwh_recv.at[zp], ccw_n).start()
            todo.append(o)
            for o in todo:
                send_block(o)

        for dz in range(N_Z):
            zp = lax.rem(z + dz, N_Z)
            o = plane_mesh(2, zp)
            mk(half(o, 0), cwh_send.at[zp], cwh_recv.at[zp], my).wait_recv()
            mk(half(o, 1), ccwh_send.at[zp], ccwh_recv.at[zp], my).wait_recv()
            send_block(o)

        def collect(o):
            pltpu.make_async_remote_copy(
                src_ref=a2a_src.at[o], dst_ref=a2a_dst.at[o],
                send_sem=a2a_send_sems.at[o], recv_sem=a2a_recv_sems.at[o],
                device_id=(my,), device_id_type=pl.DeviceIdType.MESH,
            ).wait_recv()
            out_ref[pl.ds(o * m_per, m_per), :] = a2a_dst[o].astype(jnp.float32)

        for dz in range(N_Z):
            zp = lax.rem(z + dz, N_Z)
            if dz >= 1:
                collect(c + N_C * zp)
            collect(plane_mesh(-1, zp))
            collect(plane_mesh(1, zp))
        for dz in range(N_Z):
            collect(plane_mesh(2, lax.rem(z + dz, N_Z)))

        for dz in (1, 2, 3):
            zt = lax.rem(z + dz, N_Z)
            mk(full(my), col_send.at[zt], col_recv.at[z], my).wait_send()
        for dz in range(N_Z):
            zp = lax.rem(z + dz, N_Z)
            mk(full(my), cwf_send.at[zp], cwf_recv.at[zp], my).wait_send()
            mk(full(my), ccwf_send.at[zp], ccwf_recv.at[zp], my).wait_send()
            mk(half(my, 0), cwh_send.at[zp], cwh_recv.at[zp], my).wait_send()
            mk(half(my, 1), ccwh_send.at[zp], ccwh_recv.at[zp], my).wait_send()
            if dz >= 1:
                o = c + N_C * zp
                pltpu.make_async_remote_copy(
                    src_ref=a2a_src.at[o], dst_ref=a2a_dst.at[my],
                    send_sem=a2a_send_sems.at[o],
                    recv_sem=a2a_recv_sems.at[my],
                    device_id=(my,), device_id_type=pl.DeviceIdType.MESH,
                ).wait_send()
            for dc in (-1, 1, 2):
                o = plane_mesh(dc, zp)
                pltpu.make_async_remote_copy(
                    src_ref=a2a_src.at[o], dst_ref=a2a_dst.at[my],
                    send_sem=a2a_send_sems.at[o],
                    recv_sem=a2a_recv_sems.at[my],
                    device_id=(my,), device_id_type=pl.DeviceIdType.MESH,
                ).wait_send()

    out_shape = jax.ShapeDtypeStruct((N_DEV * m_per, n_loc), jnp.float32)
    return pl.pallas_call(
        body,
        out_shape=out_shape,
        in_specs=[
            pl.BlockSpec(memory_space=pltpu.VMEM),
            pl.BlockSpec(memory_space=pltpu.VMEM),
            pl.BlockSpec(memory_space=pltpu.SMEM),
            pl.BlockSpec(memory_space=pltpu.SMEM),
        ],
        out_specs=pl.BlockSpec(memory_space=pltpu.VMEM),
        scratch_shapes=[
            pltpu.VMEM((N_DEV, k, n_loc), jnp.float8_e5m2),
            pltpu.VMEM((m_per, k), jnp.bfloat16),
            pltpu.VMEM((N_DEV, m_per, n_loc), jnp.bfloat16),
            pltpu.VMEM((N_DEV, m_per, n_loc), jnp.bfloat16),
            pltpu.SemaphoreType.DMA((N_Z,)),
            pltpu.SemaphoreType.DMA((N_Z,)),
            pltpu.SemaphoreType.DMA((N_Z,)),
            pltpu.SemaphoreType.DMA((N_Z,)),
            pltpu.SemaphoreType.DMA((N_Z,)),
            pltpu.SemaphoreType.DMA((N_Z,)),
            pltpu.SemaphoreType.DMA((N_Z,)),
            pltpu.SemaphoreType.DMA((N_Z,)),
            pltpu.SemaphoreType.DMA((N_Z,)),
            pltpu.SemaphoreType.DMA((N_Z,)),
            pltpu.SemaphoreType.DMA((N_DEV,)),
            pltpu.SemaphoreType.DMA((N_DEV,)),
        ],
        compiler_params=pltpu.CompilerParams(collective_id=0),
    )(x, w_mat, scale_x, scale_w)


# device time: 49469 ns/iter; 1.5052x vs baseline; 1.5052x over previous
import jax
import jax.numpy as jnp
from jax import lax
from jax.experimental import pallas as pl
from jax.experimental.pallas import tpu as pltpu

N_DEV = 16
HOPS = N_DEV // 2

PERM = (0, 1, 5, 9, 13, 14, 10, 6, 2, 3, 7, 11, 15, 12, 8, 4)
IPERM = tuple(PERM.index(i) for i in range(N_DEV))


def kernel(x, w_mat, scale_x, scale_w):
    m_per, k = x.shape
    n_loc = w_mat.shape[1]
    q_k = k // 4

    def body(x_ref, w_ref, sx_ref, sw_ref, perm_ref, iperm_ref, out_ref,
             wcomm, xb_ref, a2a_src, a2a_dst,
             cw_send_sems, cw_recv_sems, ccw_send_sems, ccw_recv_sems,
             a2a_send_sems, a2a_recv_sems):
        my = lax.axis_index("i")
        rp = iperm_ref[my]

        def at_ring(pos):
            return perm_ref[lax.rem(pos + 2 * N_DEV, N_DEV)]

        right = at_ring(rp + 1)
        left = at_ring(rp - 1)

        def cw_halves(s):
            return (0, 1) if s < HOPS - 1 else (0,)

        def ccw_halves(s):
            return (0, 1) if s < HOPS - 1 else (1,)

        def ring_rdma(direction, s, h, start):
            if direction == "cw":
                origin = at_ring(rp - s)
                sems, rsems, tgt = cw_send_sems, cw_recv_sems, right
            else:
                origin = at_ring(rp + s)
                sems, rsems, tgt = ccw_send_sems, ccw_recv_sems, left
            ref = wcomm.at[origin, pl.ds(h * q_k, q_k), :]
            rdma = pltpu.make_async_remote_copy(
                src_ref=ref,
                dst_ref=ref,
                send_sem=sems.at[s, h],
                recv_sem=rsems.at[s, h],
                device_id=(tgt,),
                device_id_type=pl.DeviceIdType.MESH,
            )
            if start:
                rdma.start()
            return rdma

        def ring_recv_wait(direction, j, h):
            if direction == "cw":
                origin = at_ring(rp - j - 1)
                rsems, ssems = cw_recv_sems, cw_send_sems
            else:
                origin = at_ring(rp + j + 1)
                rsems, ssems = ccw_recv_sems, ccw_send_sems
            ref = wcomm.at[origin, pl.ds(h * q_k, q_k), :]
            pltpu.make_async_remote_copy(
                src_ref=ref, dst_ref=ref,
                send_sem=ssems.at[j, h], recv_sem=rsems.at[j, h],
                device_id=(my,), device_id_type=pl.DeviceIdType.MESH,
            ).wait_recv()

        def a2a_rdma(dest, start):
            rdma = pltpu.make_async_remote_copy(
                src_ref=a2a_src.at[dest],
                dst_ref=a2a_dst.at[my],
                send_sem=a2a_send_sems.at[dest],
                recv_sem=a2a_recv_sems.at[my],
                device_id=(dest,),
                device_id_type=pl.DeviceIdType.MESH,
            )
            if start:
                rdma.start()
            return rdma

        barrier_sem = pltpu.get_barrier_semaphore()
        for t in range(N_DEV):
            pl.semaphore_signal(
                barrier_sem, inc=1,
                device_id=(t,), device_id_type=pl.DeviceIdType.MESH,
            )
        pl.semaphore_wait(barrier_sem, N_DEV)

        wcomm[my] = w_ref[...].astype(jnp.float8_e5m2)
        for h in cw_halves(0):
            ring_rdma("cw", 0, h, start=True)
        for h in ccw_halves(0):
            ring_rdma("ccw", 0, h, start=True)

        xb_ref[...] = x_ref[...].astype(jnp.bfloat16)
        scale = sx_ref[0] * sw_ref[0]

        def block(origin):
            wb = wcomm[origin].astype(jnp.bfloat16)
            acc = lax.dot_general(
                xb_ref[...], wb,
                (((1,), (0,)), ((), ())),
                preferred_element_type=jnp.float32,
            )
            y = acc * scale
            yc = jnp.clip(y, -60.0, 60.0)
            return y / (1.0 + jnp.exp(-yc))

        out_ref[pl.ds(my * m_per, m_per), :] = block(my)

        for j in range(HOPS):
            for h in (0, 1, 2, 3):
                if h in cw_halves(j):
                    ring_recv_wait("cw", j, h)
                    if j + 1 < HOPS and h in cw_halves(j + 1):
                        ring_rdma("cw", j + 1, h, start=True)
                if h in ccw_halves(j):
                    ring_recv_wait("ccw", j, h)
                    if j + 1 < HOPS and h in ccw_halves(j + 1):
                        ring_rdma("ccw", j + 1, h, start=True)
            if j < HOPS - 1:
                for origin in (at_ring(rp - j - 1), at_ring(rp + j + 1)):
                    a2a_src[origin] = block(origin).astype(jnp.bfloat16)
                    a2a_rdma(origin, start=True)
            else:
                origin = at_ring(rp + HOPS)
                a2a_src[origin] = block(origin).astype(jnp.bfloat16)
                a2a_rdma(origin, start=True)

        for d in list(range(1, HOPS + 1)) + [-dd for dd in range(1, HOPS)]:
            origin = at_ring(rp - d)
            pltpu.make_async_remote_copy(
                src_ref=a2a_src.at[origin], dst_ref=a2a_dst.at[origin],
                send_sem=a2a_send_sems.at[origin],
                recv_sem=a2a_recv_sems.at[origin],
                device_id=(my,), device_id_type=pl.DeviceIdType.MESH,
            ).wait_recv()
            out_ref[pl.ds(origin * m_per, m_per), :] = (
                a2a_dst[origin].astype(jnp.float32))

        for s in range(HOPS):
            for h in cw_halves(s):
                ring_rdma("cw", s, h, start=False).wait_send()
            for h in ccw_halves(s):
                ring_rdma("ccw", s, h, start=False).wait_send()
        for d in range(1, N_DEV):
            a2a_rdma(at_ring(rp + d), start=False).wait_send()

    out_shape = jax.ShapeDtypeStruct((N_DEV * m_per, n_loc), jnp.float32)
    return pl.pallas_call(
        body,
        out_shape=out_shape,
        in_specs=[
            pl.BlockSpec(memory_space=pltpu.VMEM),
            pl.BlockSpec(memory_space=pltpu.VMEM),
            pl.BlockSpec(memory_space=pltpu.SMEM),
            pl.BlockSpec(memory_space=pltpu.SMEM),
            pl.BlockSpec(memory_space=pltpu.SMEM),
            pl.BlockSpec(memory_space=pltpu.SMEM),
        ],
        out_specs=pl.BlockSpec(memory_space=pltpu.VMEM),
        scratch_shapes=[
            pltpu.VMEM((N_DEV, k, n_loc), jnp.float8_e5m2),
            pltpu.VMEM((m_per, k), jnp.bfloat16),
            pltpu.VMEM((N_DEV, m_per, n_loc), jnp.bfloat16),
            pltpu.VMEM((N_DEV, m_per, n_loc), jnp.bfloat16),
            pltpu.SemaphoreType.DMA((HOPS, 2)),
            pltpu.SemaphoreType.DMA((HOPS, 2)),
            pltpu.SemaphoreType.DMA((HOPS, 2)),
            pltpu.SemaphoreType.DMA((HOPS, 2)),
            pltpu.SemaphoreType.DMA((N_DEV,)),
            pltpu.SemaphoreType.DMA((N_DEV,)),
        ],
        compiler_params=pltpu.CompilerParams(collective_id=0),
    )(x, w_mat, scale_x, scale_w,
      jnp.array(PERM, jnp.int32), jnp.array(IPERM, jnp.int32))
